# baseline (device time: 106004 ns/iter reference)
import jax
import jax.numpy as jnp
from jax import lax
from jax.experimental import pallas as pl
from jax.experimental.pallas import tpu as pltpu

N_DEV = 32
NSUB = 8
COMM_DTYPE = jnp.bfloat16


def _mesh_logical_order():
    order = []
    for z in range(4):
        for y in range(4):
            row = [(0, y, z), (1, y, z)]
            if y % 2:
                row = row[::-1]
            order.extend(row)
    return order


def _hamiltonian_cycle():
    c2d = [(0, 0), (1, 0), (2, 0), (3, 0), (3, 1), (2, 1), (1, 1), (1, 2),
           (2, 2), (3, 2), (3, 3), (2, 3), (1, 3), (0, 3), (0, 2), (0, 1)]
    cyc = [(0, y, z) for (y, z) in c2d] + [(1, y, z) for (y, z) in c2d[::-1]]
    for a, b in zip(cyc, cyc[1:] + cyc[:1]):
        d = sum(abs(u - v) for u, v in zip(a, b))
        assert d == 1, (a, b)
    assert len(set(cyc)) == N_DEV
    return cyc


_LOGICAL_OF_COORD = {c: i for i, c in enumerate(_mesh_logical_order())}
_RING = [_LOGICAL_OF_COORD[c] for c in _hamiltonian_cycle()]
_POS = [0] * N_DEV
for _p, _l in enumerate(_RING):
    _POS[_l] = _p


def kernel(x, w_mat):
    m, k_sh = x.shape
    _, n = w_mat.shape
    chunk = m // N_DEV
    n_half = n // 2
    n_lane = n_half // NSUB
    n_lanes = 2 * NSUB

    ring = jnp.asarray(_RING, dtype=jnp.int32)
    pos_of = jnp.asarray(_POS, dtype=jnp.int32)
    my = lax.axis_index("i")
    p_cw = pos_of[my]
    hs = jnp.arange(N_DEV, dtype=jnp.int32)
    cs_cw = ring[(p_cw - 1 - hs) % N_DEV]
    q_ccw = (N_DEV - p_cw) % N_DEV
    cs_ccw = ring[(N_DEV - ((q_ccw - 1 - hs) % N_DEV)) % N_DEV]
    chunks = jnp.stack([cs_cw, cs_ccw]).astype(jnp.int32)
    dst_cw = ring[(p_cw + 1) % N_DEV]
    dst_ccw = ring[(p_cw - 1) % N_DEV]
    dsts = jnp.stack([dst_cw, dst_ccw]).astype(jnp.int32)

    lane_dir = [l // NSUB for l in range(n_lanes)]
    lane_col = [l * n_lane for l in range(n_lanes)]
    lane_order = [d * NSUB + s for s in range(NSUB) for d in range(2)]

    def body(dsts_ref, chunks_ref, x_ref, w_ref, out_ref, *scratch):
        send_bufs = scratch[0:n_lanes]
        recv_bufs = scratch[n_lanes:2 * n_lanes]
        send_sems = scratch[2 * n_lanes:3 * n_lanes]
        recv_sems = scratch[3 * n_lanes:4 * n_lanes]
        credit_sems = scratch[4 * n_lanes:6 * n_lanes]

        barrier_sem = pltpu.get_barrier_semaphore()
        for d in range(2):
            pl.semaphore_signal(barrier_sem, inc=1, device_id=(dsts_ref[d],),
                                device_id_type=pl.DeviceIdType.MESH)
        pl.semaphore_wait(barrier_sem, 2)

        def partial_chunk(c, col, width):
            xs = x_ref[pl.ds(c * chunk, chunk), :]
            return jnp.dot(xs, w_ref[:, col:col + width],
                           preferred_element_type=jnp.float32)

        def make_rdma(lane, slot):
            return pltpu.make_async_remote_copy(
                src_ref=send_bufs[lane].at[slot],
                dst_ref=recv_bufs[lane].at[slot],
                send_sem=send_sems[lane].at[slot],
                recv_sem=recv_sems[lane].at[slot],
                device_id=(dsts_ref[lane_dir[lane]],),
                device_id_type=pl.DeviceIdType.MESH,
            )

        rdmas = [[] for _ in range(n_lanes)]
        for h in range(N_DEV - 1):
            slot = h % 2
            pp = [partial_chunk(chunks_ref[d, h], d * n_half, n_half)
                  for d in range(2)]
            for lane in lane_order:
                d, col = lane_dir[lane], lane_col[lane]
                sub = pp[d][:, col - d * n_half:col - d * n_half + n_lane]
                prev = dsts_ref[1 - d]
                if h >= 2:
                    rdmas[lane][h - 2].wait_send()
                if h == 0:
                    send_bufs[lane][slot, :, :] = sub.astype(COMM_DTYPE)
                else:
                    pslot = (h - 1) % 2
                    rdmas[lane][h - 1].wait_recv()
                    send_bufs[lane][slot, :, :] = (
                        sub + recv_bufs[lane][pslot, :, :].astype(jnp.float32)
                    ).astype(COMM_DTYPE)
                    pl.semaphore_signal(
                        credit_sems[2 * lane + pslot], inc=1,
                        device_id=(prev,),
                        device_id_type=pl.DeviceIdType.MESH)
                if h >= 2:
                    pl.semaphore_wait(credit_sems[2 * lane + slot], 1)
                rdma = make_rdma(lane, slot)
                rdma.start()
                rdmas[lane].append(rdma)

        last = N_DEV - 2
        pp = [partial_chunk(chunks_ref[d, N_DEV - 1], d * n_half, n_half)
              for d in range(2)]
        for lane in range(n_lanes):
            d, col = lane_dir[lane], lane_col[lane]
            sub = pp[d][:, col - d * n_half:col - d * n_half + n_lane]
            rdmas[lane][last].wait_recv()
            out_ref[:, col:col + n_lane] = jnp.maximum(
                sub + recv_bufs[lane][last % 2, :, :].astype(jnp.float32), 0.0)
            rdmas[lane][last - 1].wait_send()
            rdmas[lane][last].wait_send()
            pl.semaphore_wait(credit_sems[2 * lane + (last - 1) % 2], 1)

    scratch_shapes = (
        [pltpu.VMEM((2, chunk, n_lane), COMM_DTYPE) for _ in range(n_lanes)]
        + [pltpu.VMEM((2, chunk, n_lane), COMM_DTYPE) for _ in range(n_lanes)]
        + [pltpu.SemaphoreType.DMA((2,)) for _ in range(n_lanes)]
        + [pltpu.SemaphoreType.DMA((2,)) for _ in range(n_lanes)]
        + [pltpu.SemaphoreType.REGULAR for _ in range(2 * n_lanes)]
    )
    return pl.pallas_call(
        body,
        out_shape=jax.ShapeDtypeStruct((chunk, n), jnp.float32),
        in_specs=[pl.BlockSpec(memory_space=pltpu.SMEM),
                  pl.BlockSpec(memory_space=pltpu.SMEM),
                  pl.BlockSpec(memory_space=pltpu.VMEM),
                  pl.BlockSpec(memory_space=pltpu.VMEM)],
        out_specs=pl.BlockSpec(memory_space=pltpu.VMEM),
        scratch_shapes=scratch_shapes,
        compiler_params=pltpu.CompilerParams(collective_id=0),
    )(dsts, chunks, x, w_mat)


# device time: 103578 ns/iter; 1.0234x vs baseline; 1.0234x over previous
import jax
import jax.numpy as jnp
from jax import lax
from jax.experimental import pallas as pl
from jax.experimental.pallas import tpu as pltpu

N_DEV = 32
NSUB = 4
COMM_DTYPE = jnp.bfloat16


def _mesh_logical_order():
    order = []
    for z in range(4):
        for y in range(4):
            row = [(0, y, z), (1, y, z)]
            if y % 2:
                row = row[::-1]
            order.extend(row)
    return order


def _hamiltonian_cycle():
    c2d = [(0, 0), (1, 0), (2, 0), (3, 0), (3, 1), (2, 1), (1, 1), (1, 2),
           (2, 2), (3, 2), (3, 3), (2, 3), (1, 3), (0, 3), (0, 2), (0, 1)]
    cyc = [(0, y, z) for (y, z) in c2d] + [(1, y, z) for (y, z) in c2d[::-1]]
    for a, b in zip(cyc, cyc[1:] + cyc[:1]):
        d = sum(abs(u - v) for u, v in zip(a, b))
        assert d == 1, (a, b)
    assert len(set(cyc)) == N_DEV
    return cyc


_LOGICAL_OF_COORD = {c: i for i, c in enumerate(_mesh_logical_order())}
_RING = [_LOGICAL_OF_COORD[c] for c in _hamiltonian_cycle()]
_POS = [0] * N_DEV
for _p, _l in enumerate(_RING):
    _POS[_l] = _p


def kernel(x, w_mat):
    m, k_sh = x.shape
    _, n = w_mat.shape
    chunk = m // N_DEV
    n_half = n // 2
    n_lane = n_half // NSUB
    n_lanes = 2 * NSUB

    ring = jnp.asarray(_RING, dtype=jnp.int32)
    pos_of = jnp.asarray(_POS, dtype=jnp.int32)
    my = lax.axis_index("i")
    p_cw = pos_of[my]
    hs = jnp.arange(N_DEV, dtype=jnp.int32)
    cs_cw = ring[(p_cw - 1 - hs) % N_DEV]
    q_ccw = (N_DEV - p_cw) % N_DEV
    cs_ccw = ring[(N_DEV - ((q_ccw - 1 - hs) % N_DEV)) % N_DEV]
    chunks = jnp.stack([cs_cw, cs_ccw]).astype(jnp.int32)
    dst_cw = ring[(p_cw + 1) % N_DEV]
    dst_ccw = ring[(p_cw - 1) % N_DEV]
    dsts = jnp.stack([dst_cw, dst_ccw]).astype(jnp.int32)

    lane_dir = [l // NSUB for l in range(n_lanes)]
    lane_col = [l * n_lane for l in range(n_lanes)]
    lane_order = [d * NSUB + s for s in range(NSUB) for d in range(2)]

    def body(dsts_ref, chunks_ref, x_ref, w_ref, out_ref, *scratch):
        send_bufs = scratch[0:n_lanes]
        recv_bufs = scratch[n_lanes:2 * n_lanes]
        send_sems = scratch[2 * n_lanes:3 * n_lanes]
        recv_sems = scratch[3 * n_lanes:4 * n_lanes]
        credit_sems = scratch[4 * n_lanes:4 * n_lanes + 4]

        barrier_sem = pltpu.get_barrier_semaphore()
        for d in range(2):
            pl.semaphore_signal(barrier_sem, inc=1, device_id=(dsts_ref[d],),
                                device_id_type=pl.DeviceIdType.MESH)
        pl.semaphore_wait(barrier_sem, 2)

        def partial_chunk(c, col, width):
            xs = x_ref[pl.ds(c * chunk, chunk), :]
            return jnp.dot(xs, w_ref[:, col:col + width],
                           preferred_element_type=jnp.float32)

        def make_rdma(lane, slot):
            return pltpu.make_async_remote_copy(
                src_ref=send_bufs[lane].at[slot],
                dst_ref=recv_bufs[lane].at[slot],
                send_sem=send_sems[lane].at[slot],
                recv_sem=recv_sems[lane].at[slot],
                device_id=(dsts_ref[lane_dir[lane]],),
                device_id_type=pl.DeviceIdType.MESH,
            )

        rdmas = [[] for _ in range(n_lanes)]
        for h in range(N_DEV - 1):
            slot = h % 2
            pp = [partial_chunk(chunks_ref[d, h], d * n_half, n_half)
                  for d in range(2)]
            for i, lane in enumerate(lane_order):
                d, col = lane_dir[lane], lane_col[lane]
                sub = pp[d][:, col - d * n_half:col - d * n_half + n_lane]
                prev = dsts_ref[1 - d]
                if h >= 2:
                    rdmas[lane][h - 2].wait_send()
                if h == 0:
                    send_bufs[lane][slot, :, :] = sub.astype(COMM_DTYPE)
                else:
                    pslot = (h - 1) % 2
                    rdmas[lane][h - 1].wait_recv()
                    send_bufs[lane][slot, :, :] = (
                        sub + recv_bufs[lane][pslot, :, :].astype(jnp.float32)
                    ).astype(COMM_DTYPE)
                    if i >= n_lanes - 2:
                        pl.semaphore_signal(
                            credit_sems[2 * d + pslot], inc=1,
                            device_id=(prev,),
                            device_id_type=pl.DeviceIdType.MESH)
                if h >= 2 and i < 2:
                    pl.semaphore_wait(credit_sems[2 * d + slot], 1)
                rdma = make_rdma(lane, slot)
                rdma.start()
                rdmas[lane].append(rdma)

        last = N_DEV - 2
        pp = [partial_chunk(chunks_ref[d, N_DEV - 1], d * n_half, n_half)
              for d in range(2)]
        for lane in range(n_lanes):
            d, col = lane_dir[lane], lane_col[lane]
            sub = pp[d][:, col - d * n_half:col - d * n_half + n_lane]
            rdmas[lane][last].wait_recv()
            out_ref[:, col:col + n_lane] = jnp.maximum(
                sub + recv_bufs[lane][last % 2, :, :].astype(jnp.float32), 0.0)
            rdmas[lane][last - 1].wait_send()
            rdmas[lane][last].wait_send()
        for d in range(2):
            pl.semaphore_wait(credit_sems[2 * d + (last - 1) % 2], 1)

    scratch_shapes = (
        [pltpu.VMEM((2, chunk, n_lane), COMM_DTYPE) for _ in range(n_lanes)]
        + [pltpu.VMEM((2, chunk, n_lane), COMM_DTYPE) for _ in range(n_lanes)]
        + [pltpu.SemaphoreType.DMA((2,)) for _ in range(n_lanes)]
        + [pltpu.SemaphoreType.DMA((2,)) for _ in range(n_lanes)]
        + [pltpu.SemaphoreType.REGULAR for _ in range(4)]
    )
    return pl.pallas_call(
        body,
        out_shape=jax.ShapeDtypeStruct((chunk, n), jnp.float32),
        in_specs=[pl.BlockSpec(memory_space=pltpu.SMEM),
                  pl.BlockSpec(memory_space=pltpu.SMEM),
                  pl.BlockSpec(memory_space=pltpu.VMEM),
                  pl.BlockSpec(memory_space=pltpu.VMEM)],
        out_specs=pl.BlockSpec(memory_space=pltpu.VMEM),
        scratch_shapes=scratch_shapes,
        compiler_params=pltpu.CompilerParams(collective_id=0),
    )(dsts, chunks, x, w_mat)


# device time: 99842 ns/iter; 1.0617x vs baseline; 1.0374x over previous
import jax
import jax.numpy as jnp
from jax import lax
from jax.experimental import pallas as pl
from jax.experimental.pallas import tpu as pltpu

N_DEV = 32
COMM_DTYPE = jnp.bfloat16

RING_COLS = [768, 768, 512]
SUB_W = 256


def _mesh_logical_order():
    order = []
    for z in range(4):
        for y in range(4):
            row = [(0, y, z), (1, y, z)]
            if y % 2:
                row = row[::-1]
            order.extend(row)
    return order


_CYC_A = [(0, 0, 0), (1, 0, 0), (1, 1, 0), (0, 1, 0), (0, 2, 0), (1, 2, 0),
          (1, 3, 0), (0, 3, 0), (0, 3, 1), (1, 3, 1), (1, 3, 2), (1, 3, 3),
          (0, 3, 3), (0, 3, 2), (0, 2, 2), (0, 2, 1), (1, 2, 1), (1, 2, 2),
          (1, 2, 3), (0, 2, 3), (0, 1, 3), (1, 1, 3), (1, 0, 3), (0, 0, 3),
          (0, 0, 2), (0, 1, 2), (1, 1, 2), (1, 0, 2), (1, 0, 1), (1, 1, 1),
          (0, 1, 1), (0, 0, 1)]
_CYC_B = [(1, 3, 0), (1, 3, 1), (1, 2, 1), (1, 2, 0), (1, 1, 0), (1, 1, 1),
          (1, 1, 2), (1, 1, 3), (0, 1, 3), (0, 0, 3), (1, 0, 3), (1, 0, 2),
          (0, 0, 2), (0, 0, 1), (1, 0, 1), (1, 0, 0), (0, 0, 0), (0, 1, 0),
          (0, 1, 1), (0, 1, 2), (0, 2, 2), (0, 2, 3), (0, 3, 3), (1, 3, 3),
          (1, 2, 3), (1, 2, 2), (1, 3, 2), (0, 3, 2), (0, 3, 1), (0, 2, 1),
          (0, 2, 0), (0, 3, 0)]
_CYC_C = [(0, 3, 3), (0, 2, 3), (1, 2, 3), (1, 3, 3), (1, 3, 2), (1, 3, 1),
          (1, 3, 0), (1, 2, 0), (1, 2, 1), (0, 2, 1), (0, 3, 1), (0, 3, 0),
          (0, 2, 0), (0, 1, 0), (0, 0, 0), (0, 0, 1), (0, 0, 2), (0, 0, 3),
          (0, 1, 3), (0, 1, 2), (0, 1, 1), (1, 1, 1), (1, 1, 0), (1, 0, 0),
          (1, 0, 1), (1, 0, 2), (1, 0, 3), (1, 1, 3), (1, 1, 2), (1, 2, 2),
          (0, 2, 2), (0, 3, 2)]

_CYCLES = [_CYC_A, _CYC_B, _CYC_C]

_LOGICAL_OF_COORD = {c: i for i, c in enumerate(_mesh_logical_order())}
_ALL_ARCS = set()
for _cyc in _CYCLES:
    assert len(set(_cyc)) == N_DEV
    for _a, _b in zip(_cyc, _cyc[1:] + _cyc[:1]):
        assert sum(abs(u - v) for u, v in zip(_a, _b)) == 1, (_a, _b)
        assert (_a, _b) not in _ALL_ARCS
        _ALL_ARCS.add((_a, _b))
_RINGS = [[_LOGICAL_OF_COORD[c] for c in cyc] for cyc in _CYCLES]
_POSS = []
for _r in _RINGS:
    _pos = [0] * N_DEV
    for _p, _l in enumerate(_r):
        _pos[_l] = _p
    _POSS.append(_pos)

N_RINGS = 3
_LANES = []
_off = 0
for _r in range(N_RINGS):
    for _s in range(RING_COLS[_r] // SUB_W):
        _LANES.append((_r, _off, SUB_W))
        _off += SUB_W
assert _off == sum(RING_COLS)
N_LANES = len(_LANES)
def _sub_idx(l):
    r, col, _ = _LANES[l]
    return (col - sum(RING_COLS[:r])) // SUB_W


_ORDER = sorted(range(N_LANES), key=lambda l: (_sub_idx(l), _LANES[l][0]))
_FIRST = {}
_LAST = {}
for _i, _l in enumerate(_ORDER):
    _r = _LANES[_l][0]
    _FIRST.setdefault(_r, _i)
    _LAST[_r] = _i


def kernel(x, w_mat):
    m, k_sh = x.shape
    _, n = w_mat.shape
    chunk = m // N_DEV
    ring_col0 = [sum(RING_COLS[:r]) for r in range(N_RINGS)]

    my = lax.axis_index("i")
    hs = jnp.arange(N_DEV, dtype=jnp.int32)
    cs_list, dst_list, prev_list = [], [], []
    for r in range(N_RINGS):
        ring = jnp.asarray(_RINGS[r], dtype=jnp.int32)
        pos_of = jnp.asarray(_POSS[r], dtype=jnp.int32)
        p = pos_of[my]
        cs_list.append(ring[(p - 1 - hs) % N_DEV])
        dst_list.append(ring[(p + 1) % N_DEV])
        prev_list.append(ring[(p - 1) % N_DEV])
    chunks = jnp.stack(cs_list).astype(jnp.int32)
    dsts = jnp.stack(dst_list).astype(jnp.int32)
    prevs = jnp.stack(prev_list).astype(jnp.int32)

    def body(dsts_ref, prevs_ref, chunks_ref, x_ref, w_ref, out_ref,
             *scratch):
        send_bufs = scratch[0:N_LANES]
        recv_bufs = scratch[N_LANES:2 * N_LANES]
        send_sems = scratch[2 * N_LANES:3 * N_LANES]
        recv_sems = scratch[3 * N_LANES:4 * N_LANES]
        credit_sems = scratch[4 * N_LANES:4 * N_LANES + 2 * N_RINGS]

        barrier_sem = pltpu.get_barrier_semaphore()
        for r in range(N_RINGS):
            for ref in (dsts_ref, prevs_ref):
                pl.semaphore_signal(barrier_sem, inc=1,
                                    device_id=(ref[r],),
                                    device_id_type=pl.DeviceIdType.MESH)
        pl.semaphore_wait(barrier_sem, 2 * N_RINGS)

        def partial_chunk(c, col, width):
            xs = x_ref[pl.ds(c * chunk, chunk), :]
            return jnp.dot(xs, w_ref[:, col:col + width],
                           preferred_element_type=jnp.float32)

        def make_rdma(lane, slot):
            return pltpu.make_async_remote_copy(
                src_ref=send_bufs[lane].at[slot],
                dst_ref=recv_bufs[lane].at[slot],
                send_sem=send_sems[lane].at[slot],
                recv_sem=recv_sems[lane].at[slot],
                device_id=(dsts_ref[_LANES[lane][0]],),
                device_id_type=pl.DeviceIdType.MESH,
            )

        rdmas = [[] for _ in range(N_LANES)]
        for h in range(N_DEV - 1):
            slot = h % 2
            pp = [partial_chunk(chunks_ref[r, h], ring_col0[r],
                                RING_COLS[r]) for r in range(N_RINGS)]
            for pos, i in enumerate(_ORDER):
                r, col, width = _LANES[i]
                sub = pp[r][:, col - ring_col0[r]:col - ring_col0[r] + width]
                if h >= 2:
                    rdmas[i][h - 2].wait_send()
                if h == 0:
                    send_bufs[i][slot, :, :] = sub.astype(COMM_DTYPE)
                else:
                    pslot = (h - 1) % 2
                    rdmas[i][h - 1].wait_recv()
                    send_bufs[i][slot, :, :] = (
                        sub + recv_bufs[i][pslot, :, :].astype(jnp.float32)
                    ).astype(COMM_DTYPE)
                    if pos == _LAST[r]:
                        pl.semaphore_signal(
                            credit_sems[2 * r + pslot], inc=1,
                            device_id=(prevs_ref[r],),
                            device_id_type=pl.DeviceIdType.MESH)
                if h >= 2 and pos == _FIRST[r]:
                    pl.semaphore_wait(credit_sems[2 * r + slot], 1)
                rdma = make_rdma(i, slot)
                rdma.start()
                rdmas[i].append(rdma)

        last = N_DEV - 2
        pp = [partial_chunk(chunks_ref[r, N_DEV - 1], ring_col0[r],
                            RING_COLS[r]) for r in range(N_RINGS)]
        for i in range(N_LANES):
            r, col, width = _LANES[i]
            sub = pp[r][:, col - ring_col0[r]:col - ring_col0[r] + width]
            rdmas[i][last].wait_recv()
            out_ref[:, col:col + width] = jnp.maximum(
                sub + recv_bufs[i][last % 2, :, :].astype(jnp.float32), 0.0)
            rdmas[i][last - 1].wait_send()
            rdmas[i][last].wait_send()
        for r in range(N_RINGS):
            pl.semaphore_wait(credit_sems[2 * r + (last - 1) % 2], 1)

    scratch_shapes = (
        [pltpu.VMEM((2, chunk, _LANES[i][2]), COMM_DTYPE)
         for i in range(N_LANES)]
        + [pltpu.VMEM((2, chunk, _LANES[i][2]), COMM_DTYPE)
           for i in range(N_LANES)]
        + [pltpu.SemaphoreType.DMA((2,)) for _ in range(N_LANES)]
        + [pltpu.SemaphoreType.DMA((2,)) for _ in range(N_LANES)]
        + [pltpu.SemaphoreType.REGULAR for _ in range(2 * N_RINGS)]
    )
    return pl.pallas_call(
        body,
        out_shape=jax.ShapeDtypeStruct((chunk, n), jnp.float32),
        in_specs=[pl.BlockSpec(memory_space=pltpu.SMEM),
                  pl.BlockSpec(memory_space=pltpu.SMEM),
                  pl.BlockSpec(memory_space=pltpu.SMEM),
                  pl.BlockSpec(memory_space=pltpu.VMEM),
                  pl.BlockSpec(memory_space=pltpu.VMEM)],
        out_specs=pl.BlockSpec(memory_space=pltpu.VMEM),
        scratch_shapes=scratch_shapes,
        compiler_params=pltpu.CompilerParams(collective_id=0),
    )(dsts, prevs, chunks, x, w_mat)
